# Initial kernel scaffold; baseline (speedup 1.0000x reference)
import jax
import jax.numpy as jnp
from jax.experimental import pallas as pl


def kernel(x, edge_index, W1, b1, W2, b2):
    raise NotImplementedError("write your pallas kernel here")


# speedup vs baseline: 6.8475x; 6.8475x over previous
import functools

import jax
import jax.numpy as jnp
from jax import lax
from jax.experimental import pallas as pl
from jax.experimental.pallas import tpu as pltpu
from jax.experimental.pallas import tpu_sc as plsc

_EPS = 1e-6
_N, _E, _D = 10000, 320000, 128
_BN = 1000
_CH = 128
_NCHUNK = _E // _CH
_NW = 32
_RPT = 624
_ZR = 104


def _m0(shape):
    return lax.broadcasted_iota(jnp.int32, shape, 1) == 0


def _logmap0(xa, m0):
    y = jnp.where(m0, 0.0, xa)
    y_norm = jnp.sqrt(jnp.sum(y * y, axis=-1, keepdims=True) + _EPS)
    x0 = jnp.sum(jnp.where(m0, xa, 0.0), axis=-1, keepdims=True)
    z = jnp.maximum(x0, 1.0 + _EPS)
    theta = jnp.log(z + jnp.sqrt((z - 1.0) * (z + 1.0)))
    return theta * y / y_norm


def _expmap0_proj(u, m0):
    x_norm = jnp.sqrt(jnp.sum(u * u, axis=-1, keepdims=True) + _EPS)
    et = jnp.exp(x_norm)
    sinh = 0.5 * (et - 1.0 / et)
    resr = sinh * u / x_norm
    x0 = jnp.sqrt(1.0 + jnp.sum(resr * resr, axis=-1, keepdims=True))
    return jnp.where(m0, x0, resr)


def _pre_body(x_ref, w_ref, b_ref, o_ref):
    xa = x_ref[...]
    m0 = _m0(xa.shape)
    u = _logmap0(xa, m0)
    mu = lax.dot_general(u, w_ref[...], (((1,), (1,)), ((), ())),
                         preferred_element_type=jnp.float32) + b_ref[...]
    mu = jnp.where(m0, 0.0, mu)
    h = _expmap0_proj(mu, m0)
    ht = _logmap0(h, m0)
    o_ref[...] = jnp.where(m0, 1.0, ht)


def _mid_body(p_ref, w_ref, b_ref, o_ref):
    s = p_ref[0] + p_ref[1]
    m0 = _m0(s.shape)
    deg = jnp.maximum(jnp.sum(jnp.where(m0, s, 0.0), axis=-1, keepdims=True), 1.0)
    agg = jnp.where(m0, 0.0, s / deg)
    h1 = _expmap0_proj(agg, m0)
    xt = jnp.maximum(_logmap0(h1, m0), 0.0)
    h1a = _expmap0_proj(xt, m0)
    u2 = _logmap0(h1a, m0)
    mu2 = lax.dot_general(u2, w_ref[...], (((1,), (1,)), ((), ())),
                          preferred_element_type=jnp.float32) + b_ref[...]
    mu2 = jnp.where(m0, 0.0, mu2)
    h2 = _expmap0_proj(mu2, m0)
    ht2 = _logmap0(h2, m0)
    o_ref[...] = jnp.where(m0, 1.0, ht2)


def _post_body(p_ref, o_ref):
    s = p_ref[0] + p_ref[1]
    m0 = _m0(s.shape)
    deg = jnp.maximum(jnp.sum(jnp.where(m0, s, 0.0), axis=-1, keepdims=True), 1.0)
    agg = jnp.where(m0, 0.0, s / deg)
    h = _expmap0_proj(agg, m0)
    ht = _logmap0(h, m0)
    mx = jnp.max(ht, axis=-1, keepdims=True)
    sh = ht - mx
    o_ref[...] = sh - jnp.log(jnp.sum(jnp.exp(sh), axis=-1, keepdims=True))


def _tc_pre(x, w, b):
    return pl.pallas_call(
        _pre_body,
        grid=(_N // _BN,),
        in_specs=[
            pl.BlockSpec((_BN, _D), lambda i: (i, 0)),
            pl.BlockSpec((_D, _D), lambda i: (0, 0)),
            pl.BlockSpec((1, _D), lambda i: (0, 0)),
        ],
        out_specs=pl.BlockSpec((_BN, _D), lambda i: (i, 0)),
        out_shape=jax.ShapeDtypeStruct((_N, _D), jnp.float32),
    )(x, w, b)


def _tc_mid(p, w, b):
    return pl.pallas_call(
        _mid_body,
        grid=(_N // _BN,),
        in_specs=[
            pl.BlockSpec((2, _BN, _D), lambda i: (0, i, 0)),
            pl.BlockSpec((_D, _D), lambda i: (0, 0)),
            pl.BlockSpec((1, _D), lambda i: (0, 0)),
        ],
        out_specs=pl.BlockSpec((_BN, _D), lambda i: (i, 0)),
        out_shape=jax.ShapeDtypeStruct((_N, _D), jnp.float32),
    )(p, w, b)


def _tc_post(p):
    return pl.pallas_call(
        _post_body,
        grid=(_N // _BN,),
        in_specs=[pl.BlockSpec((2, _BN, _D), lambda i: (0, i, 0))],
        out_specs=pl.BlockSpec((_BN, _D), lambda i: (i, 0)),
        out_shape=jax.ShapeDtypeStruct((_N, _D), jnp.float32),
    )(p)


def _sc_agg(ht, src, dst):
    mesh = plsc.VectorSubcoreMesh(core_axis_name="c", subcore_axis_name="s")

    @functools.partial(
        pl.kernel,
        out_type=jax.ShapeDtypeStruct((2, _N, _D), jnp.float32),
        mesh=mesh,
        scratch_types=[
            pltpu.VMEM_SHARED((_N, _D), jnp.float32),
            pltpu.VMEM((_ZR, _D), jnp.float32),
            pltpu.VMEM((_CH,), jnp.int32),
            pltpu.VMEM((_CH,), jnp.int32),
            pltpu.VMEM((_CH, _D), jnp.float32),
            pltpu.SemaphoreType.DMA,
        ],
    )
    def k(ht_hbm, src_hbm, dst_hbm, out_hbm, acc, zbuf, sidx, didx, rows, sem):
        cid = lax.axis_index("c")
        sid = lax.axis_index("s")
        wid = sid * 2 + cid

        def zrow(i, carry):
            for j in range(_D // 16):
                zbuf[i, pl.ds(j * 16, 16)] = jnp.zeros((16,), jnp.float32)
            return carry

        lax.fori_loop(0, _ZR, zrow, 0)
        row0 = sid * _RPT
        for r in range(_RPT // _ZR):
            pltpu.sync_copy(zbuf, acc.at[pl.ds(row0 + r * _ZR, _ZR)])

        @pl.when(sid == 15)
        def _():
            pltpu.sync_copy(zbuf.at[pl.ds(0, 16)], acc.at[pl.ds(_RPT * 16, 16)])

        plsc.subcore_barrier()

        lo = wid * _NCHUNK // _NW
        hi = (wid + 1) * _NCHUNK // _NW

        def body(i, carry):
            base = i * _CH
            pltpu.sync_copy(src_hbm.at[pl.ds(base, _CH)], sidx)
            pltpu.sync_copy(dst_hbm.at[pl.ds(base, _CH)], didx)
            pltpu.async_copy(ht_hbm.at[sidx], rows, sem).wait()
            pltpu.sync_copy(rows, acc.at[didx], add=True)
            return carry

        lax.fori_loop(lo, hi, body, 0)
        plsc.subcore_barrier()
        pltpu.sync_copy(acc.at[pl.ds(row0, _RPT)],
                        out_hbm.at[cid, pl.ds(row0, _RPT)])

        @pl.when(sid == 15)
        def _():
            pltpu.sync_copy(acc.at[pl.ds(_RPT * 16, 16)],
                            out_hbm.at[cid, pl.ds(_RPT * 16, 16)])

    return k(ht, src, dst)


def kernel(x, edge_index, W1, b1, W2, b2):
    src = edge_index[0]
    dst = edge_index[1]
    b1r = b1.reshape(1, _D)
    b2r = b2.reshape(1, _D)
    ht1 = _tc_pre(x, W1, b1r)
    p1 = _sc_agg(ht1, src, dst)
    ht2 = _tc_mid(p1, W2, b2r)
    p2 = _sc_agg(ht2, src, dst)
    return _tc_post(p2)
